# baseline (speedup 1.0000x reference)
import jax
import jax.numpy as jnp
from jax import lax
from jax.experimental import pallas as pl
from jax.experimental.pallas import tpu as pltpu
from jax.experimental.pallas import tpu_sc as plsc

_BINS = 256
_K = 4
_N = 65536
_C = 256
_NC = 2
_NS = 16
_NW = _NC * _NS
_NSLC = _NW // _K
_ROWS_W = _N // _NSLC
_CHUNK = 64
_NCHUNK = _ROWS_W // _CHUNK


def _sc_joint_body(p_hbm, a_hbm, out_hbm, a_all, buf0, buf1, acc, sem):
    cid = lax.axis_index("c")
    sid = lax.axis_index("s")
    wid = sid * _NC + cid
    k = wid % _K
    slc = wid // _K
    nbase = slc * _ROWS_W

    def zrow(r, _):
        for j in range(_C // 16):
            acc[r, pl.ds(j * 16, 16)] = jnp.zeros((16,), jnp.float32)
        return 0

    lax.fori_loop(0, _BINS, zrow, 0)

    pltpu.sync_copy(a_hbm.at[pl.ds(nbase, _ROWS_W)], a_all)

    bufs = (buf0, buf1)

    def start(j, slot):
        pltpu.async_copy(p_hbm.at[k, pl.ds(nbase + j * _CHUNK, _CHUNK), :],
                         bufs[slot], sem)

    def drain(slot):
        pltpu.make_async_copy(p_hbm.at[0, pl.ds(0, _CHUNK), :],
                              bufs[slot], sem).wait()

    def do_rows(buf, j):
        for g in range(_CHUNK // 16):
            av = a_all[pl.ds(j * _CHUNK + g * 16, 16)]
            for j2 in range(16):
                a_r = av[j2]
                r = g * 16 + j2
                for cb in range(_C // 16):
                    plsc.addupdate(acc.at[a_r, pl.ds(cb * 16, 16)],
                                   buf[r, pl.ds(cb * 16, 16)])

    start(0, 0)

    def pair_body(jj, _):
        for b in range(2):
            j = jj * 2 + b

            @pl.when(j + 1 < _NCHUNK)
            def _():
                start(j + 1, 1 - b)

            drain(b)
            do_rows(bufs[b], j)
        return 0

    lax.fori_loop(0, _NCHUNK // 2, pair_body, 0)

    pltpu.sync_copy(acc, out_hbm.at[slc, k])


def _sc_joint(probabilities, a32):
    return pl.kernel(
        _sc_joint_body,
        out_type=jax.ShapeDtypeStruct((_NSLC, _K, _BINS, _C), jnp.float32),
        mesh=plsc.VectorSubcoreMesh(core_axis_name="c", subcore_axis_name="s"),
        scratch_types=[
            pltpu.VMEM((_ROWS_W,), jnp.int32),
            pltpu.VMEM((_CHUNK, _C), jnp.float32),
            pltpu.VMEM((_CHUNK, _C), jnp.float32),
            pltpu.VMEM((_BINS, _C), jnp.float32),
            pltpu.SemaphoreType.DMA,
        ],
    )(probabilities, a32)


def _mi_body(part_ref, a_ref, out_ref):
    eps = 1e-10

    bins = lax.broadcasted_iota(jnp.int32, (1, 1, _BINS), 2)

    def hist_body(i, acc):
        blk = a_ref[pl.ds(i * 16, 16), :]
        oh = (blk[:, :, None] == bins).astype(jnp.float32)
        return acc + jnp.sum(oh, axis=(0, 1))

    counts = lax.fori_loop(0, _N // (16 * 256), hist_body,
                           jnp.zeros((_BINS,), jnp.float32))
    p_a = counts / jnp.sum(counts)
    h_a = -jnp.sum(p_a * jnp.log(p_a + eps))

    mi_sum = 0.0
    for k in range(_K):
        joint = part_ref[0, k]
        for s in range(1, _NSLC):
            joint = joint + part_ref[s, k]
        jp = joint / jnp.sum(joint)
        h_ab = -jnp.sum(jp * jnp.log(jp + eps))
        pb_raw = jnp.sum(joint, axis=0)
        p_b = pb_raw / jnp.sum(pb_raw)
        h_b = -jnp.sum(p_b * jnp.log(p_b + eps))
        mi_sum = mi_sum + 2.0 * (h_a + h_b - h_ab) / (h_a + h_b)
    out_ref[0, 0] = mi_sum / _K


def _mi_final(partials, a_2d):
    return pl.pallas_call(
        _mi_body,
        out_shape=jax.ShapeDtypeStruct((1, 1), jnp.float32),
        out_specs=pl.BlockSpec(memory_space=pltpu.SMEM),
    )(partials, a_2d)


def kernel(assignments, probabilities):
    a32 = assignments.astype(jnp.int32)
    partials = _sc_joint(probabilities, a32)
    out = _mi_final(partials, a32.reshape(_N // 256, 256))
    return out[0, 0]


# speedup vs baseline: 1.4276x; 1.4276x over previous
import jax
import jax.numpy as jnp
from jax import lax
from jax.experimental import pallas as pl
from jax.experimental.pallas import tpu as pltpu
from jax.experimental.pallas import tpu_sc as plsc

_BINS = 256
_K = 4
_N = 65536
_C = 256
_NC = 2
_NS = 16
_NW = _NC * _NS
_NSLC = _NW // _K
_ROWS_W = _N // _NSLC
_CHUNK = 64
_NCHUNK = _ROWS_W // _CHUNK


def _sc_joint_body(p_hbm, a_hbm, out_hbm, a_all, buf0, buf1, acc, sem):
    cid = lax.axis_index("c")
    sid = lax.axis_index("s")
    wid = sid * _NC + cid
    k = wid % _K
    slc = wid // _K
    nbase = slc * _ROWS_W

    def zrow(r, _):
        for j in range(_C // 16):
            acc[r, pl.ds(j * 16, 16)] = jnp.zeros((16,), jnp.float32)
        return 0

    lax.fori_loop(0, _BINS, zrow, 0)

    pltpu.sync_copy(a_hbm.at[pl.ds(nbase, _ROWS_W)], a_all)

    bufs = (buf0, buf1)

    def start(j, slot):
        pltpu.async_copy(p_hbm.at[k, pl.ds(nbase + j * _CHUNK, _CHUNK), :],
                         bufs[slot], sem)

    def drain(slot):
        pltpu.make_async_copy(p_hbm.at[0, pl.ds(0, _CHUNK), :],
                              bufs[slot], sem).wait()

    def do_rows(buf, j):
        for g in range(_CHUNK // 16):
            av = a_all[pl.ds(j * _CHUNK + g * 16, 16)]
            for j2 in range(16):
                a_r = av[j2]
                r = g * 16 + j2
                vals = [buf[r, pl.ds(cb * 16, 16)] for cb in range(_C // 16)]
                for cb in range(_C // 16):
                    plsc.addupdate(acc.at[a_r, pl.ds(cb * 16, 16)], vals[cb])

    start(0, 0)

    def pair_body(jj, _):
        for b in range(2):
            j = jj * 2 + b

            @pl.when(j + 1 < _NCHUNK)
            def _():
                start(j + 1, 1 - b)

            drain(b)
            do_rows(bufs[b], j)
        return 0

    lax.fori_loop(0, _NCHUNK // 2, pair_body, 0)

    pltpu.sync_copy(acc, out_hbm.at[slc, k])


def _sc_joint(probabilities, a32):
    return pl.kernel(
        _sc_joint_body,
        out_type=jax.ShapeDtypeStruct((_NSLC, _K, _BINS, _C), jnp.float32),
        mesh=plsc.VectorSubcoreMesh(core_axis_name="c", subcore_axis_name="s"),
        scratch_types=[
            pltpu.VMEM((_ROWS_W,), jnp.int32),
            pltpu.VMEM((_CHUNK, _C), jnp.float32),
            pltpu.VMEM((_CHUNK, _C), jnp.float32),
            pltpu.VMEM((_BINS, _C), jnp.float32),
            pltpu.SemaphoreType.DMA,
        ],
    )(probabilities, a32)


def _mi_body(part_ref, a_ref, out_ref):
    eps = 1e-10

    bins = lax.broadcasted_iota(jnp.int32, (1, 1, _BINS), 2)

    def hist_body(i, acc):
        blk = a_ref[pl.ds(i * 16, 16), :]
        oh = (blk[:, :, None] == bins).astype(jnp.float32)
        return acc + jnp.sum(oh, axis=(0, 1))

    counts = lax.fori_loop(0, _N // (16 * 256), hist_body,
                           jnp.zeros((_BINS,), jnp.float32))
    p_a = counts / jnp.sum(counts)
    h_a = -jnp.sum(p_a * jnp.log(p_a + eps))

    mi_sum = 0.0
    for k in range(_K):
        joint = part_ref[0, k]
        for s in range(1, _NSLC):
            joint = joint + part_ref[s, k]
        jp = joint / jnp.sum(joint)
        h_ab = -jnp.sum(jp * jnp.log(jp + eps))
        pb_raw = jnp.sum(joint, axis=0)
        p_b = pb_raw / jnp.sum(pb_raw)
        h_b = -jnp.sum(p_b * jnp.log(p_b + eps))
        mi_sum = mi_sum + 2.0 * (h_a + h_b - h_ab) / (h_a + h_b)
    out_ref[0, 0] = mi_sum / _K


def _mi_final(partials, a_2d):
    return pl.pallas_call(
        _mi_body,
        out_shape=jax.ShapeDtypeStruct((1, 1), jnp.float32),
        out_specs=pl.BlockSpec(memory_space=pltpu.SMEM),
    )(partials, a_2d)


def kernel(assignments, probabilities):
    a32 = assignments.astype(jnp.int32)
    partials = _sc_joint(probabilities, a32)
    out = _mi_final(partials, a32.reshape(_N // 256, 256))
    return out[0, 0]


# speedup vs baseline: 5.7299x; 4.0137x over previous
import jax
import jax.numpy as jnp
from jax import lax
from jax.experimental import pallas as pl
from jax.experimental.pallas import tpu as pltpu
from jax.experimental.pallas import tpu_sc as plsc

_BINS = 256
_K = 4
_N = 65536
_C = 256
_NC = 2
_NS = 16
_NW = _NC * _NS
_NSLC = _NW // _K

_CHUNK = 64
_S_TAIL = 10240
_ROWS_W = _S_TAIL // _NSLC
_NCHUNK = _ROWS_W // _CHUNK
_N_HEAD = _N - _S_TAIL
_TCBLK = 1024
_NBLK = _N_HEAD // _TCBLK


def _sc_joint_body(p_hbm, a_hbm, out_hbm, a_all, buf0, buf1, acc, sem):
    cid = lax.axis_index("c")
    sid = lax.axis_index("s")
    wid = sid * _NC + cid
    k = wid % _K
    slc = wid // _K
    nbase = _N_HEAD + slc * _ROWS_W

    def zrow(r, _):
        for j in range(_C // 16):
            acc[r, pl.ds(j * 16, 16)] = jnp.zeros((16,), jnp.float32)
        return 0

    lax.fori_loop(0, _BINS, zrow, 0)

    pltpu.sync_copy(a_hbm.at[pl.ds(nbase, _ROWS_W)], a_all)

    bufs = (buf0, buf1)

    def start(j, slot):
        pltpu.async_copy(p_hbm.at[k, pl.ds(nbase + j * _CHUNK, _CHUNK), :],
                         bufs[slot], sem)

    def drain(slot):
        pltpu.make_async_copy(p_hbm.at[0, pl.ds(0, _CHUNK), :],
                              bufs[slot], sem).wait()

    def do_rows(buf, j):
        for g in range(_CHUNK // 16):
            av = a_all[pl.ds(j * _CHUNK + g * 16, 16)]
            for j2 in range(16):
                a_r = av[j2]
                r = g * 16 + j2
                vals = [buf[r, pl.ds(cb * 16, 16)] for cb in range(_C // 16)]
                for cb in range(_C // 16):
                    plsc.addupdate(acc.at[a_r, pl.ds(cb * 16, 16)], vals[cb])

    start(0, 0)

    def pair_body(jj, _):
        for b in range(2):
            j = jj * 2 + b

            @pl.when(j + 1 < _NCHUNK)
            def _():
                start(j + 1, 1 - b)

            drain(b)
            do_rows(bufs[b], j)
        return 0

    lax.fori_loop(0, _NCHUNK // 2, pair_body, 0)

    pltpu.sync_copy(acc, out_hbm.at[slc, k])


def _sc_joint(probabilities, a32):
    return pl.kernel(
        _sc_joint_body,
        out_type=jax.ShapeDtypeStruct((_NSLC, _K, _BINS, _C), jnp.float32),
        mesh=plsc.VectorSubcoreMesh(core_axis_name="c", subcore_axis_name="s"),
        scratch_types=[
            pltpu.VMEM((_ROWS_W,), jnp.int32),
            pltpu.VMEM((_CHUNK, _C), jnp.float32),
            pltpu.VMEM((_CHUNK, _C), jnp.float32),
            pltpu.VMEM((_BINS, _C), jnp.float32),
            pltpu.SemaphoreType.DMA,
        ],
    )(probabilities, a32)


def _tc_joint_body(p_ref, a_ref, joint_ref, counts_ref, acc_ref, cnt_ref):
    i = pl.program_id(0)

    @pl.when(i == 0)
    def _():
        acc_ref[...] = jnp.zeros((_K, _BINS, _C), jnp.float32)
        cnt_ref[...] = jnp.zeros((_BINS, 1), jnp.float32)

    bins = lax.broadcasted_iota(jnp.int32, (_BINS, 1), 0)
    a_row = a_ref[0]
    cmp = (a_row == bins)
    ohT = cmp.astype(jnp.bfloat16)
    cnt_ref[...] += jnp.sum(cmp.astype(jnp.float32), axis=1, keepdims=True)
    for k in range(_K):
        pk = p_ref[k].astype(jnp.bfloat16)
        acc_ref[k] += lax.dot_general(
            ohT, pk, (((1,), (0,)), ((), ())),
            preferred_element_type=jnp.float32)

    @pl.when(i == _NBLK - 1)
    def _():
        joint_ref[...] = acc_ref[...]
        counts_ref[...] = cnt_ref[...]


def _tc_joint(probabilities, a3d):
    return pl.pallas_call(
        _tc_joint_body,
        grid=(_NBLK,),
        in_specs=[
            pl.BlockSpec((_K, _TCBLK, _C), lambda i: (0, i, 0)),
            pl.BlockSpec((1, 1, _TCBLK), lambda i: (i, 0, 0)),
        ],
        out_specs=[
            pl.BlockSpec((_K, _BINS, _C), lambda i: (0, 0, 0)),
            pl.BlockSpec((_BINS, 1), lambda i: (0, 0)),
        ],
        out_shape=[
            jax.ShapeDtypeStruct((_K, _BINS, _C), jnp.float32),
            jax.ShapeDtypeStruct((_BINS, 1), jnp.float32),
        ],
        scratch_shapes=[
            pltpu.VMEM((_K, _BINS, _C), jnp.float32),
            pltpu.VMEM((_BINS, 1), jnp.float32),
        ],
    )(probabilities, a3d)


def _mi_body(joint_ref, part_ref, counts_ref, atail_ref, out_ref):
    eps = 1e-10

    bins = lax.broadcasted_iota(jnp.int32, (1, 1, _BINS), 2)
    rows = _S_TAIL // _C

    def hist_body(i, acc):
        blk = atail_ref[pl.ds(i * 8, 8), :]
        oh = (blk[:, :, None] == bins).astype(jnp.float32)
        return acc + jnp.sum(oh, axis=(0, 1))

    counts = lax.fori_loop(0, rows // 8, hist_body,
                           counts_ref[:, 0])
    p_a = counts / jnp.sum(counts)
    h_a = -jnp.sum(p_a * jnp.log(p_a + eps))

    mi_sum = 0.0
    for k in range(_K):
        joint = joint_ref[k]
        for s in range(_NSLC):
            joint = joint + part_ref[s, k]
        jp = joint / jnp.sum(joint)
        h_ab = -jnp.sum(jp * jnp.log(jp + eps))
        pb_raw = jnp.sum(joint, axis=0)
        p_b = pb_raw / jnp.sum(pb_raw)
        h_b = -jnp.sum(p_b * jnp.log(p_b + eps))
        mi_sum = mi_sum + 2.0 * (h_a + h_b - h_ab) / (h_a + h_b)
    out_ref[0, 0] = mi_sum / _K


def _mi_final(joint, partials, counts, a_tail):
    return pl.pallas_call(
        _mi_body,
        out_shape=jax.ShapeDtypeStruct((1, 1), jnp.float32),
        out_specs=pl.BlockSpec(memory_space=pltpu.SMEM),
    )(joint, partials, counts, a_tail)


def kernel(assignments, probabilities):
    a32 = assignments.astype(jnp.int32)
    partials = _sc_joint(probabilities, a32)
    a3d = a32[:_N_HEAD].reshape(_NBLK, 1, _TCBLK)
    joint, counts = _tc_joint(probabilities, a3d)
    a_tail = a32[_N_HEAD:].reshape(_S_TAIL // _C, _C)
    out = _mi_final(joint, partials, counts, a_tail)
    return out[0, 0]
